# baseline (speedup 1.0000x reference)
import functools

import jax
import jax.numpy as jnp
from jax import lax
from jax.experimental import pallas as pl
from jax.experimental.pallas import tpu as pltpu
from jax.experimental.pallas import tpu_sc as plsc

_N = 50000
_E = 3200000
_NW = 32
_C = 1280
_NCHUNK = _E // _C
_NSLOT = 80
_NPAIR = _NSLOT // 2
_LASTB = (_NCHUNK - 1) * _C

_mesh = plsc.VectorSubcoreMesh(core_axis_name="c", subcore_axis_name="s")
_params = pltpu.CompilerParams(needs_layout_passes=False)


def _worker_id():
    return lax.axis_index("s") * 2 + lax.axis_index("c")


def _body(adj_hbm, sft_hbm, tabx_hbm, taby_hbm, tabz_hbm,
          sftx_hbm, sfty_hbm, sftz_hbm, rc2_hbm,
          adj_out_hbm, sod_hbm, part_hbm,
          tabA, tabB, sftx, sfty, sftz, rc2v,
          bw0, bs0, bp0, bw1, bs1, bp1,
          bd0, bo0, bd1, bo1,
          sin0, sin1, sout0, sout1):
    wid = _worker_id()
    pltpu.sync_copy(tabx_hbm, tabA)
    pltpu.sync_copy(taby_hbm, tabB)
    pltpu.sync_copy(sftx_hbm, sftx)
    pltpu.sync_copy(sfty_hbm, sfty)
    pltpu.sync_copy(sftz_hbm, sftz)
    pltpu.sync_copy(rc2_hbm, rc2v)
    rc2 = rc2v[...]
    neg1 = jnp.full((16,), -1, jnp.int32)

    ins = ((bw0, bs0, bp0, sin0), (bw1, bs1, bp1, sin1))
    outsA = ((bp0, sout0), (bp1, sout1))
    outsB = ((bd0, bo0, sout0), (bd1, bo1, sout1))

    def slot_base(t):
        return jnp.minimum(wid + 32 * t, _NCHUNK - 1) * _C

    def issue_in_a(s, b):
        bw, bs, _, sin = ins[s]
        pltpu.async_copy(adj_hbm.at[:, pl.ds(b, _C)], bw, sin)
        pltpu.async_copy(sft_hbm.at[pl.ds(b, _C)], bs, sin)

    def wait_in_a(s):
        bw, bs, _, sin = ins[s]
        pltpu.make_async_copy(adj_hbm.at[:, pl.ds(0, _C)], bw, sin).wait()
        pltpu.make_async_copy(sft_hbm.at[pl.ds(0, _C)], bs, sin).wait()

    def issue_in_b(s, b):
        _, _, bp, sin = ins[s]
        issue_in_a(s, b)
        pltpu.async_copy(part_hbm.at[pl.ds(b, _C)], bp, sin)

    def wait_in_b(s):
        _, _, bp, sin = ins[s]
        wait_in_a(s)
        pltpu.make_async_copy(part_hbm.at[pl.ds(0, _C)], bp, sin).wait()

    def compute_a(s):
        bw, bs, bp, _ = ins[s]

        @plsc.parallel_loop(0, _C, step=16, unroll=16)
        def _(o):
            iv = bw[0, pl.ds(o, 16)]
            jv = bw[1, pl.ds(o, 16)]
            sv = bs[pl.ds(o, 16)]
            xj = plsc.load_gather(tabA, [jv])
            xi = plsc.load_gather(tabA, [iv])
            yj = plsc.load_gather(tabB, [jv])
            yi = plsc.load_gather(tabB, [iv])
            tx = plsc.load_gather(sftx, [sv])
            ty = plsc.load_gather(sfty, [sv])
            dx = xj - xi + tx
            dy = yj - yi + ty
            bp[pl.ds(o, 16)] = dx * dx + dy * dy

    def issue_out_a(s, b):
        bp, sout = outsA[s]
        pltpu.async_copy(bp, part_hbm.at[pl.ds(b, _C)], sout)

    def wait_out_a(s):
        bp, sout = outsA[s]
        pltpu.make_async_copy(bp, part_hbm.at[pl.ds(0, _C)], sout).wait()

    issue_in_a(0, slot_base(0))

    def pair_a(t, carry):
        t0 = 2 * t
        issue_in_a(1, slot_base(t0 + 1))
        wait_in_a(0)

        @pl.when(t > 0)
        def _():
            wait_out_a(0)

        compute_a(0)
        issue_out_a(0, slot_base(t0))
        issue_in_a(0, slot_base(t0 + 2))
        wait_in_a(1)

        @pl.when(t > 0)
        def _():
            wait_out_a(1)

        compute_a(1)
        issue_out_a(1, slot_base(t0 + 1))
        return carry

    npair_w = jnp.where(wid < _NCHUNK - 32 * (_NSLOT - 2), _NPAIR,
                        _NPAIR - 1)
    lax.fori_loop(0, npair_w, pair_a, 0)
    wait_in_a(0)
    wait_out_a(0)
    wait_out_a(1)

    pltpu.sync_copy(tabz_hbm, tabA)

    def compute_b(s):
        bw, bs, bp, _ = ins[s]
        bd, bo, _ = outsB[s]

        @plsc.parallel_loop(0, _C, step=16, unroll=16)
        def _(o):
            iv = bw[0, pl.ds(o, 16)]
            jv = bw[1, pl.ds(o, 16)]
            sv = bs[pl.ds(o, 16)]
            pv = bp[pl.ds(o, 16)]
            zj = plsc.load_gather(tabA, [jv])
            zi = plsc.load_gather(tabA, [iv])
            tz = plsc.load_gather(sftz, [sv])
            dz = zj - zi + tz
            sod = pv + dz * dz
            m = sod <= rc2
            bd[pl.ds(o, 16)] = sod
            bo[0, pl.ds(o, 16)] = jnp.where(m, iv, neg1)
            bo[1, pl.ds(o, 16)] = jnp.where(m, jv, neg1)

    def issue_out_b(s, b):
        bd, bo, sout = outsB[s]
        pltpu.async_copy(bd, sod_hbm.at[pl.ds(b, _C)], sout)
        pltpu.async_copy(bo, adj_out_hbm.at[:, pl.ds(b, _C)], sout)

    def wait_out_b(s):
        bd, bo, sout = outsB[s]
        pltpu.make_async_copy(bd, sod_hbm.at[pl.ds(0, _C)], sout).wait()
        pltpu.make_async_copy(bo, adj_out_hbm.at[:, pl.ds(0, _C)],
                              sout).wait()

    issue_in_b(0, slot_base(0))

    def pair_b(t, carry):
        t0 = 2 * t
        issue_in_b(1, slot_base(t0 + 1))
        wait_in_b(0)

        @pl.when(t > 0)
        def _():
            wait_out_b(0)

        compute_b(0)
        issue_out_b(0, slot_base(t0))
        issue_in_b(0, slot_base(t0 + 2))
        wait_in_b(1)

        @pl.when(t > 0)
        def _():
            wait_out_b(1)

        compute_b(1)
        issue_out_b(1, slot_base(t0 + 1))
        return carry

    lax.fori_loop(0, npair_w, pair_b, 0)
    wait_in_b(0)
    wait_out_b(0)
    wait_out_b(1)


_run = functools.partial(
    pl.kernel,
    out_type=(jax.ShapeDtypeStruct((2, _E), jnp.int32),
              jax.ShapeDtypeStruct((_E,), jnp.float32),
              jax.ShapeDtypeStruct((_E,), jnp.float32)),
    mesh=_mesh,
    compiler_params=_params,
    scratch_types=(
        [pltpu.VMEM((_N,), jnp.float32)] * 2
        + [pltpu.VMEM((32,), jnp.float32)] * 3
        + [pltpu.VMEM((16,), jnp.float32)]
        + [pltpu.VMEM((2, _C), jnp.int32), pltpu.VMEM((_C,), jnp.int32),
           pltpu.VMEM((_C,), jnp.float32)]
        + [pltpu.VMEM((2, _C), jnp.int32), pltpu.VMEM((_C,), jnp.int32),
           pltpu.VMEM((_C,), jnp.float32)]
        + [pltpu.VMEM((_C,), jnp.float32), pltpu.VMEM((2, _C), jnp.int32)]
        + [pltpu.VMEM((_C,), jnp.float32), pltpu.VMEM((2, _C), jnp.int32)]
        + [pltpu.SemaphoreType.DMA] * 4
    ),
)(_body)


def kernel(pos_xyz, cel_mat, sft_cel, spc, adj_ij, sft_idx, rc):
    cel = cel_mat[0]
    pos = pos_xyz[0] + spc[0].astype(jnp.float32) @ cel
    sftm = sft_cel @ cel
    sft_pad = jnp.pad(sftm, ((0, 5), (0, 0)))
    rc_f = jnp.asarray(rc, jnp.float32)
    rc2v = jnp.full((16,), rc_f * rc_f, jnp.float32)

    tabx = jnp.copy(pos[:, 0])
    taby = jnp.copy(pos[:, 1])
    tabz = jnp.copy(pos[:, 2])
    sftx = jnp.copy(sft_pad[:, 0])
    sfty = jnp.copy(sft_pad[:, 1])
    sftz = jnp.copy(sft_pad[:, 2])

    adj_out, sod, _unused_part = _run(
        adj_ij, sft_idx, tabx, taby, tabz, sftx, sfty, sftz, rc2v)
    return adj_out, sod


# speedup vs baseline: 1.0204x; 1.0204x over previous
import functools

import jax
import jax.numpy as jnp
from jax import lax
from jax.experimental import pallas as pl
from jax.experimental.pallas import tpu as pltpu
from jax.experimental.pallas import tpu_sc as plsc

_N = 50000
_E = 3200000
_NW = 32
_C = 1280
_NCHUNK = _E // _C
_NSLOT = 80
_NPAIR = _NSLOT // 2
_LASTB = (_NCHUNK - 1) * _C

_mesh = plsc.VectorSubcoreMesh(core_axis_name="c", subcore_axis_name="s")
_params = pltpu.CompilerParams(needs_layout_passes=False)


def _worker_id():
    return lax.axis_index("s") * 2 + lax.axis_index("c")


def _body(adj_hbm, sft_hbm, tabx_hbm, taby_hbm, tabz_hbm,
          sftx_hbm, sfty_hbm, sftz_hbm, rc2_hbm,
          adj_out_hbm, sod_hbm, part_hbm,
          tabA, tabB, sftx, sfty, sftz, rc2v,
          bw0, bs0, bp0, bw1, bs1, bp1,
          bd0, bo0, bd1, bo1,
          sin0, sin1, sout0, sout1):
    wid = _worker_id()
    pltpu.sync_copy(tabx_hbm, tabA)
    pltpu.sync_copy(taby_hbm, tabB)
    pltpu.sync_copy(sftx_hbm, sftx)
    pltpu.sync_copy(sfty_hbm, sfty)
    pltpu.sync_copy(sftz_hbm, sftz)
    pltpu.sync_copy(rc2_hbm, rc2v)
    rc2 = rc2v[...]
    neg1 = jnp.full((16,), -1, jnp.int32)

    ins = ((bw0, bs0, bp0, sin0), (bw1, bs1, bp1, sin1))
    outsA = ((bp0, sout0), (bp1, sout1))
    outsB = ((bd0, bo0, sout0), (bd1, bo1, sout1))

    def slot_base(t):
        return jnp.minimum(wid + 32 * t, _NCHUNK - 1) * _C

    def issue_in_a(s, b):
        bw, bs, _, sin = ins[s]
        pltpu.async_copy(adj_hbm.at[:, pl.ds(b, _C)], bw, sin)
        pltpu.async_copy(sft_hbm.at[pl.ds(b, _C)], bs, sin)

    def wait_in_a(s):
        bw, bs, _, sin = ins[s]
        pltpu.make_async_copy(adj_hbm.at[:, pl.ds(0, _C)], bw, sin).wait()
        pltpu.make_async_copy(sft_hbm.at[pl.ds(0, _C)], bs, sin).wait()

    def issue_in_b(s, b):
        _, _, bp, sin = ins[s]
        issue_in_a(s, b)
        pltpu.async_copy(part_hbm.at[pl.ds(b, _C)], bp, sin)

    def wait_in_b(s):
        _, _, bp, sin = ins[s]
        wait_in_a(s)
        pltpu.make_async_copy(part_hbm.at[pl.ds(0, _C)], bp, sin).wait()

    def compute_a(s):
        bw, bs, bp, _ = ins[s]

        @plsc.parallel_loop(0, _C, step=16, unroll=8)
        def _(o):
            iv = bw[0, pl.ds(o, 16)]
            jv = bw[1, pl.ds(o, 16)]
            sv = bs[pl.ds(o, 16)]
            xj = plsc.load_gather(tabA, [jv])
            xi = plsc.load_gather(tabA, [iv])
            yj = plsc.load_gather(tabB, [jv])
            yi = plsc.load_gather(tabB, [iv])
            tx = plsc.load_gather(sftx, [sv])
            ty = plsc.load_gather(sfty, [sv])
            dx = xj - xi + tx
            dy = yj - yi + ty
            bp[pl.ds(o, 16)] = dx * dx + dy * dy

    def issue_out_a(s, b):
        bp, sout = outsA[s]
        pltpu.async_copy(bp, part_hbm.at[pl.ds(b, _C)], sout)

    def wait_out_a(s):
        bp, sout = outsA[s]
        pltpu.make_async_copy(bp, part_hbm.at[pl.ds(0, _C)], sout).wait()

    issue_in_a(0, slot_base(0))

    def pair_a(t, carry):
        t0 = 2 * t
        issue_in_a(1, slot_base(t0 + 1))
        wait_in_a(0)

        @pl.when(t > 0)
        def _():
            wait_out_a(0)

        compute_a(0)
        issue_out_a(0, slot_base(t0))
        issue_in_a(0, slot_base(t0 + 2))
        wait_in_a(1)

        @pl.when(t > 0)
        def _():
            wait_out_a(1)

        compute_a(1)
        issue_out_a(1, slot_base(t0 + 1))
        return carry

    npair_w = jnp.where(wid < _NCHUNK - 32 * (_NSLOT - 2), _NPAIR,
                        _NPAIR - 1)
    lax.fori_loop(0, npair_w, pair_a, 0)
    wait_in_a(0)
    wait_out_a(0)
    wait_out_a(1)

    pltpu.sync_copy(tabz_hbm, tabA)

    def compute_b(s):
        bw, bs, bp, _ = ins[s]
        bd, bo, _ = outsB[s]

        @plsc.parallel_loop(0, _C, step=16, unroll=8)
        def _(o):
            iv = bw[0, pl.ds(o, 16)]
            jv = bw[1, pl.ds(o, 16)]
            sv = bs[pl.ds(o, 16)]
            pv = bp[pl.ds(o, 16)]
            zj = plsc.load_gather(tabA, [jv])
            zi = plsc.load_gather(tabA, [iv])
            tz = plsc.load_gather(sftz, [sv])
            dz = zj - zi + tz
            sod = pv + dz * dz
            m = sod <= rc2
            bd[pl.ds(o, 16)] = sod
            bo[0, pl.ds(o, 16)] = jnp.where(m, iv, neg1)
            bo[1, pl.ds(o, 16)] = jnp.where(m, jv, neg1)

    def issue_out_b(s, b):
        bd, bo, sout = outsB[s]
        pltpu.async_copy(bd, sod_hbm.at[pl.ds(b, _C)], sout)
        pltpu.async_copy(bo, adj_out_hbm.at[:, pl.ds(b, _C)], sout)

    def wait_out_b(s):
        bd, bo, sout = outsB[s]
        pltpu.make_async_copy(bd, sod_hbm.at[pl.ds(0, _C)], sout).wait()
        pltpu.make_async_copy(bo, adj_out_hbm.at[:, pl.ds(0, _C)],
                              sout).wait()

    issue_in_b(0, slot_base(0))

    def pair_b(t, carry):
        t0 = 2 * t
        issue_in_b(1, slot_base(t0 + 1))
        wait_in_b(0)

        @pl.when(t > 0)
        def _():
            wait_out_b(0)

        compute_b(0)
        issue_out_b(0, slot_base(t0))
        issue_in_b(0, slot_base(t0 + 2))
        wait_in_b(1)

        @pl.when(t > 0)
        def _():
            wait_out_b(1)

        compute_b(1)
        issue_out_b(1, slot_base(t0 + 1))
        return carry

    lax.fori_loop(0, npair_w, pair_b, 0)
    wait_in_b(0)
    wait_out_b(0)
    wait_out_b(1)


_run = functools.partial(
    pl.kernel,
    out_type=(jax.ShapeDtypeStruct((2, _E), jnp.int32),
              jax.ShapeDtypeStruct((_E,), jnp.float32),
              jax.ShapeDtypeStruct((_E,), jnp.float32)),
    mesh=_mesh,
    compiler_params=_params,
    scratch_types=(
        [pltpu.VMEM((_N,), jnp.float32)] * 2
        + [pltpu.VMEM((32,), jnp.float32)] * 3
        + [pltpu.VMEM((16,), jnp.float32)]
        + [pltpu.VMEM((2, _C), jnp.int32), pltpu.VMEM((_C,), jnp.int32),
           pltpu.VMEM((_C,), jnp.float32)]
        + [pltpu.VMEM((2, _C), jnp.int32), pltpu.VMEM((_C,), jnp.int32),
           pltpu.VMEM((_C,), jnp.float32)]
        + [pltpu.VMEM((_C,), jnp.float32), pltpu.VMEM((2, _C), jnp.int32)]
        + [pltpu.VMEM((_C,), jnp.float32), pltpu.VMEM((2, _C), jnp.int32)]
        + [pltpu.SemaphoreType.DMA] * 4
    ),
)(_body)


def kernel(pos_xyz, cel_mat, sft_cel, spc, adj_ij, sft_idx, rc):
    cel = cel_mat[0]
    pos = pos_xyz[0] + spc[0].astype(jnp.float32) @ cel
    sftm = sft_cel @ cel
    sft_pad = jnp.pad(sftm, ((0, 5), (0, 0)))
    rc_f = jnp.asarray(rc, jnp.float32)
    rc2v = jnp.full((16,), rc_f * rc_f, jnp.float32)

    tabx = jnp.copy(pos[:, 0])
    taby = jnp.copy(pos[:, 1])
    tabz = jnp.copy(pos[:, 2])
    sftx = jnp.copy(sft_pad[:, 0])
    sfty = jnp.copy(sft_pad[:, 1])
    sftz = jnp.copy(sft_pad[:, 2])

    adj_out, sod, _unused_part = _run(
        adj_ij, sft_idx, tabx, taby, tabz, sftx, sfty, sftz, rc2v)
    return adj_out, sod
